# baseline (speedup 1.0000x reference)
import functools

import jax
import jax.numpy as jnp
from jax import lax
from jax.experimental import pallas as pl
from jax.experimental.pallas import tpu as pltpu
from jax.experimental.pallas import tpu_sc as plsc

D_MODEL = 64

_NC = 2
_NS = 16
_NW = _NC * _NS
_OW = 128
_NB = 4
_GA = 2


def _pair_gather(table, idx_even, idx_odd):
    P = idx_even.shape[0]
    p_per_w = P // _NW
    n_chunks = p_per_w // _OW
    mesh = plsc.VectorSubcoreMesh(core_axis_name="c", subcore_axis_name="s")

    @functools.partial(
        pl.kernel,
        mesh=mesh,
        compiler_params=pltpu.CompilerParams(use_tc_tiling_on_sc=False),
        out_type=jax.ShapeDtypeStruct((P, 2 * D_MODEL), jnp.float32),
        scratch_types=[
            pltpu.VMEM((p_per_w,), jnp.int32),
            pltpu.VMEM((p_per_w,), jnp.int32),
            pltpu.VMEM((_NB, _OW, D_MODEL), jnp.float32),
            pltpu.VMEM((_NB, _OW, D_MODEL), jnp.float32),
            pltpu.SemaphoreType.DMA((_NB,)),
            pltpu.SemaphoreType.DMA((_NB,)),
            pltpu.SemaphoreType.DMA((_NB,)),
            pltpu.SemaphoreType.DMA((_NB,)),
        ],
    )
    def k(table_hbm, idxe_hbm, idxo_hbm, out_hbm,
          idxe_v, idxo_v, rowse_v, rowso_v, gseme, gsemo, wseme, wsemo):
        wid = lax.axis_index("s") * _NC + lax.axis_index("c")
        base = wid * p_per_w
        pltpu.sync_copy(idxe_hbm.at[pl.ds(base, p_per_w)], idxe_v)
        pltpu.sync_copy(idxo_hbm.at[pl.ds(base, p_per_w)], idxo_v)

        def issue_gather(chunk, buf):
            pltpu.async_copy(
                table_hbm.at[idxe_v.at[pl.ds(chunk * _OW, _OW)]],
                rowse_v.at[buf],
                gseme.at[buf],
            )
            pltpu.async_copy(
                table_hbm.at[idxo_v.at[pl.ds(chunk * _OW, _OW)]],
                rowso_v.at[buf],
                gsemo.at[buf],
            )

        def wait_gather(buf):
            pltpu.make_async_copy(
                table_hbm.at[pl.ds(0, _OW)], rowse_v.at[buf], gseme.at[buf]
            ).wait()
            pltpu.make_async_copy(
                table_hbm.at[pl.ds(0, _OW)], rowso_v.at[buf], gsemo.at[buf]
            ).wait()

        def issue_write(chunk, buf):
            rows = pl.ds(base + chunk * _OW, _OW)
            pltpu.async_copy(
                rowse_v.at[buf],
                out_hbm.at[rows, pl.ds(0, D_MODEL)],
                wseme.at[buf],
            )
            pltpu.async_copy(
                rowso_v.at[buf],
                out_hbm.at[rows, pl.ds(D_MODEL, D_MODEL)],
                wsemo.at[buf],
            )

        def wait_write(buf):
            pltpu.make_async_copy(
                rowse_v.at[buf],
                out_hbm.at[pl.ds(0, _OW), pl.ds(0, D_MODEL)],
                wseme.at[buf],
            ).wait()
            pltpu.make_async_copy(
                rowso_v.at[buf],
                out_hbm.at[pl.ds(0, _OW), pl.ds(D_MODEL, D_MODEL)],
                wsemo.at[buf],
            ).wait()

        for j in range(_GA):
            issue_gather(j, j)

        def body(i, carry):
            b = lax.rem(i, _NB)
            wait_gather(b)
            issue_write(i, b)
            nxt = i + _GA

            @pl.when(nxt < n_chunks)
            def _():
                bn = lax.rem(nxt, _NB)

                @pl.when(nxt >= _NB)
                def _():
                    wait_write(bn)

                issue_gather(nxt, bn)

            return carry

        lax.fori_loop(0, n_chunks, body, 0)

        for j in range(_NB):
            wait_write(j)

    return k(table, idx_even, idx_odd)


def kernel(indices, pe):
    b0, b1 = indices.shape
    flat = indices.reshape(b0 * b1).astype(jnp.int32)
    out = _pair_gather(pe, flat[0::2], flat[1::2])
    return out.reshape(b0, b1, D_MODEL)
